# Initial kernel scaffold; baseline (speedup 1.0000x reference)
import jax
import jax.numpy as jnp
from jax.experimental import pallas as pl


def kernel(x, table):
    raise NotImplementedError("write your pallas kernel here")


# speedup vs baseline: 4.8029x; 4.8029x over previous
import functools

import jax
import jax.numpy as jnp
from jax import lax
from jax.experimental import pallas as pl
from jax.experimental.pallas import tpu as pltpu
from jax.experimental.pallas import tpu_sc as plsc

_NC = 2
_NS = 16
_NW = _NC * _NS

_DIM = 32
_CHUNK = 1024


@functools.partial(jax.jit, static_argnames=("n_chunks",))
def _embed_gather(x_flat, table, *, n_chunks):
    b = x_flat.shape[0]
    b_per_w = b // _NW

    mesh = plsc.VectorSubcoreMesh(core_axis_name="c", subcore_axis_name="s")

    @functools.partial(
        pl.kernel,
        mesh=mesh,
        out_type=jax.ShapeDtypeStruct((b, _DIM), jnp.float32),
        scratch_types=[
            pltpu.VMEM((_CHUNK,), jnp.int32),
            pltpu.VMEM((_CHUNK, _DIM), jnp.float32),
            pltpu.SemaphoreType.DMA,
        ],
        compiler_params=pltpu.CompilerParams(use_tc_tiling_on_sc=False),
    )
    def k(x_hbm, table_hbm, out_hbm, idx_v, rows_v, gsem):
        wid = lax.axis_index("s") * _NC + lax.axis_index("c")
        base = wid * b_per_w

        def body(i, _):
            off = base + i * _CHUNK
            pltpu.sync_copy(x_hbm.at[pl.ds(off, _CHUNK)], idx_v)
            pltpu.async_copy(table_hbm.at[idx_v], rows_v, gsem).wait()
            pltpu.sync_copy(rows_v, out_hbm.at[pl.ds(off, _CHUNK)])
            return 0

        lax.fori_loop(0, n_chunks, body, 0)

    return k(x_flat, table)


def kernel(x, table):
    bsz, seq = x.shape
    b = bsz * seq
    x_flat = x.reshape(b).astype(jnp.int32)
    n_chunks = b // (_NW * _CHUNK)
    out = _embed_gather(x_flat, table, n_chunks=n_chunks)
    return out.reshape(bsz, seq, _DIM)


# speedup vs baseline: 5.0445x; 1.0503x over previous
import functools

import jax
import jax.numpy as jnp
from jax import lax
from jax.experimental import pallas as pl
from jax.experimental.pallas import tpu as pltpu
from jax.experimental.pallas import tpu_sc as plsc

_NC = 2
_NS = 16
_NW = _NC * _NS

_DIM = 32
_CHUNK = 1024
_NBUF = 3


@functools.partial(jax.jit, static_argnames=("n_chunks",))
def _embed_gather(x_flat, table, *, n_chunks):
    b = x_flat.shape[0]
    b_per_w = b // _NW

    mesh = plsc.VectorSubcoreMesh(core_axis_name="c", subcore_axis_name="s")

    @functools.partial(
        pl.kernel,
        mesh=mesh,
        out_type=jax.ShapeDtypeStruct((b, _DIM), jnp.float32),
        scratch_types=[
            pltpu.VMEM((_NBUF, _CHUNK), jnp.int32),
            pltpu.VMEM((_NBUF, _CHUNK, _DIM), jnp.float32),
            pltpu.SemaphoreType.DMA((_NBUF,)),
            pltpu.SemaphoreType.DMA((_NBUF,)),
        ],
        compiler_params=pltpu.CompilerParams(use_tc_tiling_on_sc=False),
    )
    def k(x_hbm, table_hbm, out_hbm, idx_v, rows_v, gsem, osem):
        wid = lax.axis_index("s") * _NC + lax.axis_index("c")
        base = wid * b_per_w

        def start_gather(i, bi):
            off = base + i * _CHUNK
            pltpu.sync_copy(x_hbm.at[pl.ds(off, _CHUNK)], idx_v.at[bi])
            pltpu.async_copy(
                table_hbm.at[idx_v.at[bi]], rows_v.at[bi], gsem.at[bi]
            )

        def wait_gather(bi):
            pltpu.make_async_copy(
                table_hbm.at[idx_v.at[bi]], rows_v.at[bi], gsem.at[bi]
            ).wait()

        def start_store(i, bi):
            off = base + i * _CHUNK
            pltpu.async_copy(
                rows_v.at[bi], out_hbm.at[pl.ds(off, _CHUNK)], osem.at[bi]
            )

        def wait_store(bi):
            pltpu.make_async_copy(
                rows_v.at[bi], out_hbm.at[pl.ds(base, _CHUNK)], osem.at[bi]
            ).wait()

        start_gather(0, 0)

        def body(i, _):
            bi = lax.rem(i, _NBUF)
            bp = lax.rem(i + (_NBUF - 1), _NBUF)

            @pl.when(i >= _NBUF)
            def _():
                wait_store(bi)

            start_gather(i, bi)
            wait_gather(bp)
            start_store(i - 1, bp)
            return 0

        lax.fori_loop(1, n_chunks, body, 0)

        last = n_chunks - 1
        wait_gather(last % _NBUF)
        pltpu.sync_copy(
            rows_v.at[last % _NBUF],
            out_hbm.at[pl.ds(base + last * _CHUNK, _CHUNK)],
        )
        for j in range(max(n_chunks - _NBUF, 0), last):
            wait_store(j % _NBUF)

    return k(x_flat, table)


def kernel(x, table):
    bsz, seq = x.shape
    b = bsz * seq
    x_flat = x.reshape(b).astype(jnp.int32)
    n_chunks = b // (_NW * _CHUNK)
    out = _embed_gather(x_flat, table, n_chunks=n_chunks)
    return out.reshape(bsz, seq, _DIM)
